# baseline (speedup 1.0000x reference)
import functools

import jax
import jax.numpy as jnp
from jax import lax
from jax.experimental import pallas as pl
from jax.experimental.pallas import tpu as pltpu
from jax.experimental.pallas import tpu_sc as plsc

N_NODES = 10000
D_H = 128
D_OUT = 2
BN_EPS = 1e-5

NC = 2
NS = 16
NW = NC * NS
K = 128
NPAD = 10240
RPT = NPAD // NS
ZR = 64
B_ROWS = 1000

_MESH = dict(core_axis_name="c", subcore_axis_name="s")


@functools.lru_cache(maxsize=None)
def _sc_agg(dcols: int, cpw: int):
    assert cpw % 2 == 0

    def body(table, src, dst, out, sidx0, didx0, sidx1, didx1,
             rows0, rows1, zbuf, acc, sg0, sg1, si1):
        c = lax.axis_index("c")
        s = lax.axis_index("s")
        wid = s * NC + c

        def _zrow(i, carry):
            for j in range(dcols // 16):
                zbuf[i, pl.ds(j * 16, 16)] = jnp.zeros((16,), jnp.float32)
            return carry

        lax.fori_loop(0, ZR, _zrow, 0)
        for t in range(RPT // ZR):
            pltpu.sync_copy(zbuf, acc.at[pl.ds(s * RPT + t * ZR, ZR)])
        plsc.subcore_barrier()

        def _chunk(j, carry):
            base = (wid * cpw + j) * K
            pltpu.sync_copy(src.at[pl.ds(base, K)], sidx0)
            pltpu.sync_copy(dst.at[pl.ds(base, K)], didx0)
            pltpu.async_copy(table.at[sidx0], rows0, sg0).wait()
            pltpu.sync_copy(rows0, acc.at[didx0], add=True)
            return carry

        lax.fori_loop(0, cpw, _chunk, 0)
        plsc.subcore_barrier()
        pltpu.sync_copy(acc.at[pl.ds(s * RPT, RPT)],
                        out.at[c].at[pl.ds(s * RPT, RPT)])

    return pl.kernel(
        body,
        out_type=jax.ShapeDtypeStruct((NC, NPAD, dcols), jnp.float32),
        mesh=plsc.VectorSubcoreMesh(**_MESH),
        scratch_types=[
            pltpu.VMEM((K,), jnp.int32),
            pltpu.VMEM((K,), jnp.int32),
            pltpu.VMEM((K,), jnp.int32),
            pltpu.VMEM((K,), jnp.int32),
            pltpu.VMEM((K, dcols), jnp.float32),
            pltpu.VMEM((K, dcols), jnp.float32),
            pltpu.VMEM((ZR, dcols), jnp.float32),
            pltpu.VMEM_SHARED((NPAD, dcols), jnp.float32),
            pltpu.SemaphoreType.DMA,
            pltpu.SemaphoreType.DMA,
            pltpu.SemaphoreType.DMA,
        ],
    )


@functools.lru_cache(maxsize=None)
def _sc_deg(cpw: int):
    def body(dst, out, didx, ones, zbuf, acc):
        c = lax.axis_index("c")
        s = lax.axis_index("s")
        wid = s * NC + c

        def _fill(i, carry):
            for j in range(D_H // 16):
                zbuf[i, pl.ds(j * 16, 16)] = jnp.zeros((16,), jnp.float32)
                ones[i, pl.ds(j * 16, 16)] = jnp.full((16,), 1.0,
                                                      jnp.float32)
            return carry

        lax.fori_loop(0, ZR, _fill, 0)

        def _fill2(i, carry):
            for j in range(D_H // 16):
                ones[i, pl.ds(j * 16, 16)] = jnp.full((16,), 1.0,
                                                      jnp.float32)
            return carry

        lax.fori_loop(ZR, K, _fill2, 0)
        for t in range(RPT // ZR):
            pltpu.sync_copy(zbuf, acc.at[pl.ds(s * RPT + t * ZR, ZR)])
        plsc.subcore_barrier()

        def _chunk(j, carry):
            base = (wid * cpw + j) * K
            pltpu.sync_copy(dst.at[pl.ds(base, K)], didx)
            pltpu.sync_copy(ones, acc.at[didx], add=True)
            return carry

        lax.fori_loop(0, cpw, _chunk, 0)
        plsc.subcore_barrier()
        pltpu.sync_copy(acc.at[pl.ds(s * RPT, RPT)],
                        out.at[c].at[pl.ds(s * RPT, RPT)])

    return pl.kernel(
        body,
        out_type=jax.ShapeDtypeStruct((NC, NPAD, D_H), jnp.float32),
        mesh=plsc.VectorSubcoreMesh(**_MESH),
        scratch_types=[
            pltpu.VMEM((K,), jnp.int32),
            pltpu.VMEM((K, D_H), jnp.float32),
            pltpu.VMEM((ZR, D_H), jnp.float32),
            pltpu.VMEM_SHARED((NPAD, D_H), jnp.float32),
        ],
    )


def _dinv_of(degp_ref):
    deg = degp_ref[0, :, 0:1] + degp_ref[1, :, 0:1]
    return jnp.where(deg > 0, lax.rsqrt(deg), 0.0)


def _t_first(x, W1, degp):
    def body(x_ref, w_ref, d_ref, o_ref):
        dinv = _dinv_of(d_ref)
        o_ref[...] = jnp.dot(x_ref[...], w_ref[...], precision=lax.Precision.HIGHEST,
                             preferred_element_type=jnp.float32) * dinv

    n = x.shape[0]
    grid = n // B_ROWS
    return pl.pallas_call(
        body,
        grid=(grid,),
        in_specs=[
            pl.BlockSpec((B_ROWS, D_H), lambda i: (i, 0)),
            pl.BlockSpec((D_H, D_H), lambda i: (0, 0)),
            pl.BlockSpec((NC, B_ROWS, D_H), lambda i: (0, i, 0)),
        ],
        out_specs=pl.BlockSpec((B_ROWS, D_H), lambda i: (i, 0)),
        out_shape=jax.ShapeDtypeStruct((n, D_H), jnp.float32),
    )(x, W1, degp)


def _t_mid(aggp, degp, b, g, be, Wn):
    dn = Wn.shape[1]

    def body(a_ref, d_ref, b_ref, g_ref, be_ref, w_ref, o_ref):
        dinv = _dinv_of(d_ref)
        a = (a_ref[0] + a_ref[1]) * dinv + b_ref[...]
        gs = g_ref[...] * lax.rsqrt(jnp.float32(1.0 + BN_EPS))
        h = jnp.maximum(a * gs + be_ref[...], 0.0)
        o_ref[...] = jnp.dot(h, w_ref[...], precision=lax.Precision.HIGHEST,
                             preferred_element_type=jnp.float32) * dinv

    grid = N_NODES // B_ROWS
    return pl.pallas_call(
        body,
        grid=(grid,),
        in_specs=[
            pl.BlockSpec((NC, B_ROWS, D_H), lambda i: (0, i, 0)),
            pl.BlockSpec((NC, B_ROWS, D_H), lambda i: (0, i, 0)),
            pl.BlockSpec((1, D_H), lambda i: (0, 0)),
            pl.BlockSpec((1, D_H), lambda i: (0, 0)),
            pl.BlockSpec((1, D_H), lambda i: (0, 0)),
            pl.BlockSpec((D_H, dn), lambda i: (0, 0)),
        ],
        out_specs=pl.BlockSpec((B_ROWS, dn), lambda i: (i, 0)),
        out_shape=jax.ShapeDtypeStruct((N_NODES, dn), jnp.float32),
    )(aggp, degp, b, g, be, Wn)


def _t_final(aggp, degp, b4p):
    def body(a_ref, d_ref, b_ref, o_ref):
        dinv = _dinv_of(d_ref)
        z = (a_ref[0] + a_ref[1]) * dinv + b_ref[...]
        colmask = lax.broadcasted_iota(jnp.int32, z.shape, 1) < D_OUT
        zm = jnp.where(colmask, z, -1e30)
        m = jnp.max(zm, axis=1, keepdims=True)
        e = jnp.where(colmask, jnp.exp(z - m), 0.0)
        lse = m + jnp.log(jnp.sum(e, axis=1, keepdims=True))
        o_ref[...] = z - lse

    grid = N_NODES // B_ROWS
    return pl.pallas_call(
        body,
        grid=(grid,),
        in_specs=[
            pl.BlockSpec((NC, B_ROWS, D_H), lambda i: (0, i, 0)),
            pl.BlockSpec((NC, B_ROWS, D_H), lambda i: (0, i, 0)),
            pl.BlockSpec((1, D_H), lambda i: (0, 0)),
        ],
        out_specs=pl.BlockSpec((B_ROWS, D_H), lambda i: (i, 0)),
        out_shape=jax.ShapeDtypeStruct((N_NODES, D_H), jnp.float32),
    )(aggp, degp, b4p)


def kernel(x, edge_index, W1, b1, gamma1, beta1, W2, b2, gamma2, beta2,
           W3, b3, gamma3, beta3, W4, b4):
    ei = edge_index.astype(jnp.int32)
    loops = jnp.arange(N_NODES, dtype=jnp.int32)
    src = jnp.concatenate([ei[0], loops])
    dst = jnp.concatenate([ei[1], loops])
    e = src.shape[0]
    cpw = -(-e // (NW * K))
    cpw += cpw % 2
    epad = cpw * NW * K
    src = jnp.concatenate([src, jnp.zeros((epad - e,), jnp.int32)])
    dst = jnp.concatenate([dst, jnp.full((epad - e,), N_NODES, jnp.int32)])

    degp = _sc_deg(cpw)(dst)

    b1r, g1r, be1r = b1[None, :], gamma1[None, :], beta1[None, :]
    b2r, g2r, be2r = b2[None, :], gamma2[None, :], beta2[None, :]
    b3r, g3r, be3r = b3[None, :], gamma3[None, :], beta3[None, :]
    W4p = jnp.zeros((D_H, D_H), jnp.float32).at[:, :D_OUT].set(W4)
    b4p = jnp.zeros((1, D_H), jnp.float32).at[0, :D_OUT].set(b4)

    y = _t_first(x, W1, degp)
    aggp = _sc_agg(D_H, cpw)(y, src, dst)
    y = _t_mid(aggp, degp, b1r, g1r, be1r, W2)
    aggp = _sc_agg(D_H, cpw)(y, src, dst)
    y = _t_mid(aggp, degp, b2r, g2r, be2r, W3)
    aggp = _sc_agg(D_H, cpw)(y, src, dst)
    y = _t_mid(aggp, degp, b3r, g3r, be3r, W4p)
    aggp = _sc_agg(D_H, cpw)(y, src, dst)
    z = _t_final(aggp, degp, b4p)
    return z[:, :D_OUT]


# speedup vs baseline: 1.2581x; 1.2581x over previous
import functools

import jax
import jax.numpy as jnp
from jax import lax
from jax.experimental import pallas as pl
from jax.experimental.pallas import tpu as pltpu
from jax.experimental.pallas import tpu_sc as plsc

N_NODES = 10000
D_H = 128
D_OUT = 2
BN_EPS = 1e-5

NC = 2
NS = 16
NW = NC * NS
K = 128
NPAD = 10240
RPT = NPAD // NS
ZR = 64
B_ROWS = 1000

_MESH = dict(core_axis_name="c", subcore_axis_name="s")


@functools.lru_cache(maxsize=None)
def _sc_agg(dcols: int, cpw: int):
    assert cpw % 2 == 0

    def body(table, src, dst, out, sidx0, didx0, sidx1, didx1,
             rows0, rows1, zbuf, acc, sg0, sg1, si1):
        c = lax.axis_index("c")
        s = lax.axis_index("s")
        wid = s * NC + c

        def _zrow(i, carry):
            for j in range(dcols // 16):
                zbuf[i, pl.ds(j * 16, 16)] = jnp.zeros((16,), jnp.float32)
            return carry

        lax.fori_loop(0, ZR, _zrow, 0)
        for t in range(RPT // ZR):
            pltpu.sync_copy(zbuf, acc.at[pl.ds(s * RPT + t * ZR, ZR)])
        plsc.subcore_barrier()

        base0 = wid * cpw * K
        pltpu.sync_copy(src.at[pl.ds(base0, K)], sidx0)
        pltpu.sync_copy(dst.at[pl.ds(base0, K)], didx0)
        pltpu.async_copy(table.at[sidx0], rows0, sg0)
        pltpu.async_copy(src.at[pl.ds(base0 + K, K)], sidx1, si1)
        pltpu.async_copy(dst.at[pl.ds(base0 + K, K)], didx1, si1)

        def _pair(p, carry):
            j0 = 2 * p
            b1 = (wid * cpw + j0 + 1) * K
            pltpu.make_async_copy(src.at[pl.ds(b1, K)], sidx1, si1).wait()
            pltpu.make_async_copy(dst.at[pl.ds(b1, K)], didx1, si1).wait()
            pltpu.async_copy(table.at[sidx1], rows1, sg1)
            pltpu.make_async_copy(table.at[sidx0], rows0, sg0).wait()
            pltpu.sync_copy(rows0, acc.at[didx0], add=True)

            @pl.when(j0 + 2 < cpw)
            def _():
                b2 = (wid * cpw + j0 + 2) * K
                pltpu.sync_copy(src.at[pl.ds(b2, K)], sidx0)
                pltpu.sync_copy(dst.at[pl.ds(b2, K)], didx0)
                pltpu.async_copy(table.at[sidx0], rows0, sg0)

            pltpu.make_async_copy(table.at[sidx1], rows1, sg1).wait()
            pltpu.sync_copy(rows1, acc.at[didx1], add=True)

            @pl.when(j0 + 3 < cpw)
            def _():
                b3 = (wid * cpw + j0 + 3) * K
                pltpu.async_copy(src.at[pl.ds(b3, K)], sidx1, si1)
                pltpu.async_copy(dst.at[pl.ds(b3, K)], didx1, si1)

            return carry

        lax.fori_loop(0, cpw // 2, _pair, 0)
        plsc.subcore_barrier()
        pltpu.sync_copy(acc.at[pl.ds(s * RPT, RPT)],
                        out.at[c].at[pl.ds(s * RPT, RPT)])

    return pl.kernel(
        body,
        out_type=jax.ShapeDtypeStruct((NC, NPAD, dcols), jnp.float32),
        mesh=plsc.VectorSubcoreMesh(**_MESH),
        scratch_types=[
            pltpu.VMEM((K,), jnp.int32),
            pltpu.VMEM((K,), jnp.int32),
            pltpu.VMEM((K,), jnp.int32),
            pltpu.VMEM((K,), jnp.int32),
            pltpu.VMEM((K, dcols), jnp.float32),
            pltpu.VMEM((K, dcols), jnp.float32),
            pltpu.VMEM((ZR, dcols), jnp.float32),
            pltpu.VMEM_SHARED((NPAD, dcols), jnp.float32),
            pltpu.SemaphoreType.DMA,
            pltpu.SemaphoreType.DMA,
            pltpu.SemaphoreType.DMA,
        ],
    )


@functools.lru_cache(maxsize=None)
def _sc_deg(cpw: int):
    def body(dst, out, didx, ones, zbuf, acc):
        c = lax.axis_index("c")
        s = lax.axis_index("s")
        wid = s * NC + c

        def _fill(i, carry):
            for j in range(D_H // 16):
                zbuf[i, pl.ds(j * 16, 16)] = jnp.zeros((16,), jnp.float32)
                ones[i, pl.ds(j * 16, 16)] = jnp.full((16,), 1.0,
                                                      jnp.float32)
            return carry

        lax.fori_loop(0, ZR, _fill, 0)

        def _fill2(i, carry):
            for j in range(D_H // 16):
                ones[i, pl.ds(j * 16, 16)] = jnp.full((16,), 1.0,
                                                      jnp.float32)
            return carry

        lax.fori_loop(ZR, K, _fill2, 0)
        for t in range(RPT // ZR):
            pltpu.sync_copy(zbuf, acc.at[pl.ds(s * RPT + t * ZR, ZR)])
        plsc.subcore_barrier()

        def _chunk(j, carry):
            base = (wid * cpw + j) * K
            pltpu.sync_copy(dst.at[pl.ds(base, K)], didx)
            pltpu.sync_copy(ones, acc.at[didx], add=True)
            return carry

        lax.fori_loop(0, cpw, _chunk, 0)
        plsc.subcore_barrier()
        pltpu.sync_copy(acc.at[pl.ds(s * RPT, RPT)],
                        out.at[c].at[pl.ds(s * RPT, RPT)])

    return pl.kernel(
        body,
        out_type=jax.ShapeDtypeStruct((NC, NPAD, D_H), jnp.float32),
        mesh=plsc.VectorSubcoreMesh(**_MESH),
        scratch_types=[
            pltpu.VMEM((K,), jnp.int32),
            pltpu.VMEM((K, D_H), jnp.float32),
            pltpu.VMEM((ZR, D_H), jnp.float32),
            pltpu.VMEM_SHARED((NPAD, D_H), jnp.float32),
        ],
    )


def _dinv_of(degp_ref):
    deg = degp_ref[0, :, 0:1] + degp_ref[1, :, 0:1]
    return jnp.where(deg > 0, lax.rsqrt(deg), 0.0)


def _t_first(x, W1, degp):
    def body(x_ref, w_ref, d_ref, o_ref):
        dinv = _dinv_of(d_ref)
        o_ref[...] = jnp.dot(x_ref[...], w_ref[...], precision=lax.Precision.HIGHEST,
                             preferred_element_type=jnp.float32) * dinv

    n = x.shape[0]
    grid = n // B_ROWS
    return pl.pallas_call(
        body,
        grid=(grid,),
        in_specs=[
            pl.BlockSpec((B_ROWS, D_H), lambda i: (i, 0)),
            pl.BlockSpec((D_H, D_H), lambda i: (0, 0)),
            pl.BlockSpec((NC, B_ROWS, D_H), lambda i: (0, i, 0)),
        ],
        out_specs=pl.BlockSpec((B_ROWS, D_H), lambda i: (i, 0)),
        out_shape=jax.ShapeDtypeStruct((n, D_H), jnp.float32),
    )(x, W1, degp)


def _t_mid(aggp, degp, b, g, be, Wn):
    dn = Wn.shape[1]

    def body(a_ref, d_ref, b_ref, g_ref, be_ref, w_ref, o_ref):
        dinv = _dinv_of(d_ref)
        a = (a_ref[0] + a_ref[1]) * dinv + b_ref[...]
        gs = g_ref[...] * lax.rsqrt(jnp.float32(1.0 + BN_EPS))
        h = jnp.maximum(a * gs + be_ref[...], 0.0)
        o_ref[...] = jnp.dot(h, w_ref[...], precision=lax.Precision.HIGHEST,
                             preferred_element_type=jnp.float32) * dinv

    grid = N_NODES // B_ROWS
    return pl.pallas_call(
        body,
        grid=(grid,),
        in_specs=[
            pl.BlockSpec((NC, B_ROWS, D_H), lambda i: (0, i, 0)),
            pl.BlockSpec((NC, B_ROWS, D_H), lambda i: (0, i, 0)),
            pl.BlockSpec((1, D_H), lambda i: (0, 0)),
            pl.BlockSpec((1, D_H), lambda i: (0, 0)),
            pl.BlockSpec((1, D_H), lambda i: (0, 0)),
            pl.BlockSpec((D_H, dn), lambda i: (0, 0)),
        ],
        out_specs=pl.BlockSpec((B_ROWS, dn), lambda i: (i, 0)),
        out_shape=jax.ShapeDtypeStruct((N_NODES, dn), jnp.float32),
    )(aggp, degp, b, g, be, Wn)


def _t_final(aggp, degp, b4p):
    def body(a_ref, d_ref, b_ref, o_ref):
        dinv = _dinv_of(d_ref)
        z = (a_ref[0] + a_ref[1]) * dinv + b_ref[...]
        colmask = lax.broadcasted_iota(jnp.int32, z.shape, 1) < D_OUT
        zm = jnp.where(colmask, z, -1e30)
        m = jnp.max(zm, axis=1, keepdims=True)
        e = jnp.where(colmask, jnp.exp(z - m), 0.0)
        lse = m + jnp.log(jnp.sum(e, axis=1, keepdims=True))
        o_ref[...] = z - lse

    grid = N_NODES // B_ROWS
    return pl.pallas_call(
        body,
        grid=(grid,),
        in_specs=[
            pl.BlockSpec((NC, B_ROWS, D_H), lambda i: (0, i, 0)),
            pl.BlockSpec((NC, B_ROWS, D_H), lambda i: (0, i, 0)),
            pl.BlockSpec((1, D_H), lambda i: (0, 0)),
        ],
        out_specs=pl.BlockSpec((B_ROWS, D_H), lambda i: (i, 0)),
        out_shape=jax.ShapeDtypeStruct((N_NODES, D_H), jnp.float32),
    )(aggp, degp, b4p)


def kernel(x, edge_index, W1, b1, gamma1, beta1, W2, b2, gamma2, beta2,
           W3, b3, gamma3, beta3, W4, b4):
    ei = edge_index.astype(jnp.int32)
    loops = jnp.arange(N_NODES, dtype=jnp.int32)
    src = jnp.concatenate([ei[0], loops])
    dst = jnp.concatenate([ei[1], loops])
    e = src.shape[0]
    cpw = -(-e // (NW * K))
    cpw += cpw % 2
    epad = cpw * NW * K
    src = jnp.concatenate([src, jnp.zeros((epad - e,), jnp.int32)])
    dst = jnp.concatenate([dst, jnp.full((epad - e,), N_NODES, jnp.int32)])

    degp = _sc_deg(cpw)(dst)

    b1r, g1r, be1r = b1[None, :], gamma1[None, :], beta1[None, :]
    b2r, g2r, be2r = b2[None, :], gamma2[None, :], beta2[None, :]
    b3r, g3r, be3r = b3[None, :], gamma3[None, :], beta3[None, :]
    W4p = jnp.zeros((D_H, D_H), jnp.float32).at[:, :D_OUT].set(W4)
    b4p = jnp.zeros((1, D_H), jnp.float32).at[0, :D_OUT].set(b4)

    y = _t_first(x, W1, degp)
    aggp = _sc_agg(D_H, cpw)(y, src, dst)
    y = _t_mid(aggp, degp, b1r, g1r, be1r, W2)
    aggp = _sc_agg(D_H, cpw)(y, src, dst)
    y = _t_mid(aggp, degp, b2r, g2r, be2r, W3)
    aggp = _sc_agg(D_H, cpw)(y, src, dst)
    y = _t_mid(aggp, degp, b3r, g3r, be3r, W4p)
    aggp = _sc_agg(D_H, cpw)(y, src, dst)
    z = _t_final(aggp, degp, b4p)
    return z[:, :D_OUT]


# speedup vs baseline: 2.3302x; 1.8521x over previous
import functools

import jax
import jax.numpy as jnp
from jax import lax
from jax.experimental import pallas as pl
from jax.experimental.pallas import tpu as pltpu
from jax.experimental.pallas import tpu_sc as plsc

N_NODES = 10000
D_H = 128
D_OUT = 2
BN_EPS = 1e-5

NC = 2
NS = 16
NW = NC * NS
K = 128
NPAD = 10240
RPT = NPAD // NS
ZR = 64
B_ROWS = 1000

_MESH = dict(core_axis_name="c", subcore_axis_name="s")


@functools.lru_cache(maxsize=None)
def _sc_agg(dcols: int, cpw0: int, cpw1: int):
    assert cpw0 % 2 == 0 and cpw1 % 2 == 0

    def body(table, src, dst, out, sidx0, didx0, sidx1, didx1,
             rows0, rows1, zbuf, acc, sg0, sg1, si1):
        c = lax.axis_index("c")
        s = lax.axis_index("s")

        def _zrow(i, carry):
            for j in range(dcols // 16):
                zbuf[i, pl.ds(j * 16, 16)] = jnp.zeros((16,), jnp.float32)
            return carry

        lax.fori_loop(0, ZR, _zrow, 0)
        for t in range(RPT // ZR):
            pltpu.sync_copy(zbuf, acc.at[pl.ds(s * RPT + t * ZR, ZR)])
        plsc.subcore_barrier()

        def _run(cnt, chunk0):
            base0 = chunk0 * K
            pltpu.sync_copy(src.at[pl.ds(base0, K)], sidx0)
            pltpu.sync_copy(dst.at[pl.ds(base0, K)], didx0)
            pltpu.async_copy(table.at[sidx0], rows0, sg0)
            pltpu.async_copy(src.at[pl.ds(base0 + K, K)], sidx1, si1)
            pltpu.async_copy(dst.at[pl.ds(base0 + K, K)], didx1, si1)

            def _pair(p, carry):
                j0 = 2 * p
                b1 = (chunk0 + j0 + 1) * K
                pltpu.make_async_copy(src.at[pl.ds(b1, K)], sidx1,
                                      si1).wait()
                pltpu.make_async_copy(dst.at[pl.ds(b1, K)], didx1,
                                      si1).wait()
                pltpu.async_copy(table.at[sidx1], rows1, sg1)
                pltpu.make_async_copy(table.at[sidx0], rows0, sg0).wait()
                pltpu.sync_copy(rows0, acc.at[didx0], add=True)

                @pl.when(j0 + 2 < cnt)
                def _():
                    b2 = (chunk0 + j0 + 2) * K
                    pltpu.sync_copy(src.at[pl.ds(b2, K)], sidx0)
                    pltpu.sync_copy(dst.at[pl.ds(b2, K)], didx0)
                    pltpu.async_copy(table.at[sidx0], rows0, sg0)

                pltpu.make_async_copy(table.at[sidx1], rows1, sg1).wait()
                pltpu.sync_copy(rows1, acc.at[didx1], add=True)

                @pl.when(j0 + 3 < cnt)
                def _():
                    b3 = (chunk0 + j0 + 3) * K
                    pltpu.async_copy(src.at[pl.ds(b3, K)], sidx1, si1)
                    pltpu.async_copy(dst.at[pl.ds(b3, K)], didx1, si1)

                return carry

            lax.fori_loop(0, cnt // 2, _pair, 0)

        @pl.when(c == 0)
        def _():
            _run(cpw0, s * cpw0)

        if cpw1 > 0:
            @pl.when(c == 1)
            def _():
                _run(cpw1, NS * cpw0 + s * cpw1)

        plsc.subcore_barrier()
        pltpu.sync_copy(acc.at[pl.ds(s * RPT, RPT)],
                        out.at[c].at[pl.ds(s * RPT, RPT)])

    return pl.kernel(
        body,
        out_type=jax.ShapeDtypeStruct((NC, NPAD, dcols), jnp.float32),
        mesh=plsc.VectorSubcoreMesh(**_MESH),
        scratch_types=[
            pltpu.VMEM((K,), jnp.int32),
            pltpu.VMEM((K,), jnp.int32),
            pltpu.VMEM((K,), jnp.int32),
            pltpu.VMEM((K,), jnp.int32),
            pltpu.VMEM((K, dcols), jnp.float32),
            pltpu.VMEM((K, dcols), jnp.float32),
            pltpu.VMEM((ZR, dcols), jnp.float32),
            pltpu.VMEM_SHARED((NPAD, dcols), jnp.float32),
            pltpu.SemaphoreType.DMA,
            pltpu.SemaphoreType.DMA,
            pltpu.SemaphoreType.DMA,
        ],
    )


@functools.lru_cache(maxsize=None)
def _sc_deg(cpw: int):
    def body(dst, out, didx, ones, zbuf, acc):
        c = lax.axis_index("c")
        s = lax.axis_index("s")
        wid = s * NC + c

        def _fill(i, carry):
            for j in range(D_H // 16):
                zbuf[i, pl.ds(j * 16, 16)] = jnp.zeros((16,), jnp.float32)
                ones[i, pl.ds(j * 16, 16)] = jnp.full((16,), 1.0,
                                                      jnp.float32)
            return carry

        lax.fori_loop(0, ZR, _fill, 0)

        def _fill2(i, carry):
            for j in range(D_H // 16):
                ones[i, pl.ds(j * 16, 16)] = jnp.full((16,), 1.0,
                                                      jnp.float32)
            return carry

        lax.fori_loop(ZR, K, _fill2, 0)
        for t in range(RPT // ZR):
            pltpu.sync_copy(zbuf, acc.at[pl.ds(s * RPT + t * ZR, ZR)])
        plsc.subcore_barrier()

        def _chunk(j, carry):
            base = (wid * cpw + j) * K
            pltpu.sync_copy(dst.at[pl.ds(base, K)], didx)
            pltpu.sync_copy(ones, acc.at[didx], add=True)
            return carry

        lax.fori_loop(0, cpw, _chunk, 0)
        plsc.subcore_barrier()
        pltpu.sync_copy(acc.at[pl.ds(s * RPT, RPT)],
                        out.at[c].at[pl.ds(s * RPT, RPT)])

    return pl.kernel(
        body,
        out_type=jax.ShapeDtypeStruct((NC, NPAD, D_H), jnp.float32),
        mesh=plsc.VectorSubcoreMesh(**_MESH),
        scratch_types=[
            pltpu.VMEM((K,), jnp.int32),
            pltpu.VMEM((K, D_H), jnp.float32),
            pltpu.VMEM((ZR, D_H), jnp.float32),
            pltpu.VMEM_SHARED((NPAD, D_H), jnp.float32),
        ],
    )


def _dinv_of(degp_ref):
    deg = degp_ref[0, :, 0:1] + degp_ref[1, :, 0:1]
    return jnp.where(deg > 0, lax.rsqrt(deg), 0.0)


def _t_first(x, W1, degp):
    def body(x_ref, w_ref, d_ref, o_ref):
        dinv = _dinv_of(d_ref)
        o_ref[...] = jnp.dot(x_ref[...], w_ref[...], precision=lax.Precision.HIGHEST,
                             preferred_element_type=jnp.float32) * dinv

    n = x.shape[0]
    grid = n // B_ROWS
    return pl.pallas_call(
        body,
        grid=(grid,),
        in_specs=[
            pl.BlockSpec((B_ROWS, D_H), lambda i: (i, 0)),
            pl.BlockSpec((D_H, D_H), lambda i: (0, 0)),
            pl.BlockSpec((NC, B_ROWS, D_H), lambda i: (0, i, 0)),
        ],
        out_specs=pl.BlockSpec((B_ROWS, D_H), lambda i: (i, 0)),
        out_shape=jax.ShapeDtypeStruct((n, D_H), jnp.float32),
    )(x, W1, degp)


def _t_mid(aggp, degp, b, g, be, Wn):
    dn = Wn.shape[1]

    def body(a_ref, d_ref, b_ref, g_ref, be_ref, w_ref, o_ref):
        dinv = _dinv_of(d_ref)
        a = (a_ref[0] + a_ref[1]) * dinv + b_ref[...]
        gs = g_ref[...] * lax.rsqrt(jnp.float32(1.0 + BN_EPS))
        h = jnp.maximum(a * gs + be_ref[...], 0.0)
        o_ref[...] = jnp.dot(h, w_ref[...], precision=lax.Precision.HIGHEST,
                             preferred_element_type=jnp.float32) * dinv

    grid = N_NODES // B_ROWS
    return pl.pallas_call(
        body,
        grid=(grid,),
        in_specs=[
            pl.BlockSpec((NC, B_ROWS, D_H), lambda i: (0, i, 0)),
            pl.BlockSpec((NC, B_ROWS, D_H), lambda i: (0, i, 0)),
            pl.BlockSpec((1, D_H), lambda i: (0, 0)),
            pl.BlockSpec((1, D_H), lambda i: (0, 0)),
            pl.BlockSpec((1, D_H), lambda i: (0, 0)),
            pl.BlockSpec((D_H, dn), lambda i: (0, 0)),
        ],
        out_specs=pl.BlockSpec((B_ROWS, dn), lambda i: (i, 0)),
        out_shape=jax.ShapeDtypeStruct((N_NODES, dn), jnp.float32),
    )(aggp, degp, b, g, be, Wn)


def _t_final(aggp, degp, b4p):
    def body(a_ref, d_ref, b_ref, o_ref):
        dinv = _dinv_of(d_ref)
        z = (a_ref[0] + a_ref[1]) * dinv + b_ref[...]
        colmask = lax.broadcasted_iota(jnp.int32, z.shape, 1) < D_OUT
        zm = jnp.where(colmask, z, -1e30)
        m = jnp.max(zm, axis=1, keepdims=True)
        e = jnp.where(colmask, jnp.exp(z - m), 0.0)
        lse = m + jnp.log(jnp.sum(e, axis=1, keepdims=True))
        o_ref[...] = z - lse

    grid = N_NODES // B_ROWS
    return pl.pallas_call(
        body,
        grid=(grid,),
        in_specs=[
            pl.BlockSpec((NC, B_ROWS, D_H), lambda i: (0, i, 0)),
            pl.BlockSpec((NC, B_ROWS, D_H), lambda i: (0, i, 0)),
            pl.BlockSpec((1, D_H), lambda i: (0, 0)),
        ],
        out_specs=pl.BlockSpec((B_ROWS, D_H), lambda i: (i, 0)),
        out_shape=jax.ShapeDtypeStruct((N_NODES, D_H), jnp.float32),
    )(aggp, degp, b4p)


def kernel(x, edge_index, W1, b1, gamma1, beta1, W2, b2, gamma2, beta2,
           W3, b3, gamma3, beta3, W4, b4):
    ei = edge_index.astype(jnp.int32)
    loops = jnp.arange(N_NODES, dtype=jnp.int32)
    src = jnp.concatenate([ei[0], loops])
    dst = jnp.concatenate([ei[1], loops])
    e = src.shape[0]
    cpd = -(-e // (NW * K))
    epad = cpd * NW * K
    tpc = 2 * cpd
    cpw0 = int(round(tpc * 0.74 / 2)) * 2
    cpw1 = tpc - cpw0
    src = jnp.concatenate([src, jnp.zeros((epad - e,), jnp.int32)])
    dst = jnp.concatenate([dst, jnp.full((epad - e,), N_NODES, jnp.int32)])

    degp = _sc_deg(cpd)(dst)

    b1r, g1r, be1r = b1[None, :], gamma1[None, :], beta1[None, :]
    b2r, g2r, be2r = b2[None, :], gamma2[None, :], beta2[None, :]
    b3r, g3r, be3r = b3[None, :], gamma3[None, :], beta3[None, :]
    W4p = jnp.zeros((D_H, D_H), jnp.float32).at[:, :D_OUT].set(W4)
    b4p = jnp.zeros((1, D_H), jnp.float32).at[0, :D_OUT].set(b4)

    y = _t_first(x, W1, degp)
    aggp = _sc_agg(D_H, cpw0, cpw1)(y, src, dst)
    y = _t_mid(aggp, degp, b1r, g1r, be1r, W2)
    aggp = _sc_agg(D_H, cpw0, cpw1)(y, src, dst)
    y = _t_mid(aggp, degp, b2r, g2r, be2r, W3)
    aggp = _sc_agg(D_H, cpw0, cpw1)(y, src, dst)
    y = _t_mid(aggp, degp, b3r, g3r, be3r, W4p)
    aggp = _sc_agg(D_H, cpw0, cpw1)(y, src, dst)
    z = _t_final(aggp, degp, b4p)
    return z[:, :D_OUT]
